# baseline (speedup 1.0000x reference)
import functools

import jax
import jax.numpy as jnp
from jax import lax
from jax.experimental import pallas as pl
from jax.experimental.pallas import tpu as pltpu
from jax.experimental.pallas import tpu_sc as plsc

B, L, V, D = 4096, 200, 100000, 128
LP = 208
LANES = 16
NC, NS = 2, 16
NW = NC * NS
BPW = B // NW
LH = LP // 2
UNROLL = 8
NCH = D // LANES


def _sc_sums(emb, idxm2):
    mesh = plsc.VectorSubcoreMesh(core_axis_name="c", subcore_axis_name="s")

    @functools.partial(
        pl.kernel,
        out_type=jax.ShapeDtypeStruct((B, D), jnp.float32),
        mesh=mesh,
        scratch_types=[
            pltpu.VMEM((2 * BPW, LH), jnp.int32),
            pltpu.VMEM((LP, D), jnp.float32),
            pltpu.VMEM((LP, D), jnp.float32),
            pltpu.VMEM((BPW, D), jnp.float32),
            pltpu.SemaphoreType.DMA,
            pltpu.SemaphoreType.DMA,
        ],
    )
    def k(emb_hbm, idx_hbm, out_hbm, idx_v, rows0, rows1, out_v, sem0, sem1):
        wid = lax.axis_index("s") * NC + lax.axis_index("c")
        base = wid * BPW
        pltpu.sync_copy(idx_hbm.at[pl.ds(2 * base, 2 * BPW)], idx_v)

        def issue(b, rows, sem):
            pltpu.async_copy(emb_hbm.at[idx_v.at[2 * b]],
                             rows.at[pl.ds(0, LH)], sem)
            pltpu.async_copy(emb_hbm.at[idx_v.at[2 * b + 1]],
                             rows.at[pl.ds(LH, LH)], sem)

        def drain(b, rows, sem):
            pltpu.make_async_copy(emb_hbm.at[idx_v.at[2 * b]],
                                  rows.at[pl.ds(0, LH)], sem).wait()
            pltpu.make_async_copy(emb_hbm.at[idx_v.at[2 * b + 1]],
                                  rows.at[pl.ds(LH, LH)], sem).wait()

        def accum(b, rows):
            def rowstep(j, acc):
                r0 = j * UNROLL
                for u in range(UNROLL):
                    acc = tuple(acc[c] + rows[r0 + u, pl.ds(c * LANES, LANES)]
                                for c in range(NCH))
                return acc

            acc = lax.fori_loop(
                0, LP // UNROLL, rowstep,
                tuple(jnp.zeros((LANES,), jnp.float32) for _ in range(NCH)))
            for c in range(NCH):
                out_v[b, pl.ds(c * LANES, LANES)] = acc[c]

        issue(0, rows0, sem0)

        def pair(g, carry):
            b0 = 2 * g
            b1 = 2 * g + 1
            issue(b1, rows1, sem1)
            drain(b0, rows0, sem0)
            accum(b0, rows0)
            issue(lax.rem(b0 + 2, BPW), rows0, sem0)
            drain(b1, rows1, sem1)
            accum(b1, rows1)
            return carry

        lax.fori_loop(0, BPW // 2, pair, 0)
        drain(0, rows0, sem0)
        pltpu.sync_copy(out_v, out_hbm.at[pl.ds(base, BPW)])

    return k(emb, idxm2)


def _tc_finish(sums, mask, proj_t, emb0):
    BB = 512

    def body(s_ref, m_ref, p_ref, e0_ref, o_ref):
        cnt = jnp.sum(m_ref[...].astype(jnp.float32), axis=1, keepdims=True)
        corr = s_ref[...] - (LP - cnt) * e0_ref[...]
        y = jnp.dot(corr, p_ref[...], preferred_element_type=jnp.float32)
        o_ref[...] = y / jnp.clip(cnt, 1e-9, None)

    return pl.pallas_call(
        body,
        grid=(B // BB,),
        in_specs=[
            pl.BlockSpec((BB, D), lambda i: (i, 0)),
            pl.BlockSpec((BB, L), lambda i: (i, 0)),
            pl.BlockSpec((D, D), lambda i: (0, 0)),
            pl.BlockSpec((1, D), lambda i: (0, 0)),
        ],
        out_specs=pl.BlockSpec((BB, D), lambda i: (i, 0)),
        out_shape=jax.ShapeDtypeStruct((B, D), jnp.float32),
    )(sums, mask, proj_t, emb0)


def kernel(input_ids, attention_mask, emb, proj):
    ids = input_ids.astype(jnp.int32)
    msk = attention_mask.astype(jnp.int32)
    idxm = jnp.pad(ids * msk, ((0, 0), (0, LP - L)))
    sums = _sc_sums(emb, idxm.reshape(2 * B, LH))
    return _tc_finish(sums, msk, proj.T, emb[0:1])


# speedup vs baseline: 66.2361x; 66.2361x over previous
import functools

import jax
import jax.numpy as jnp
from jax import lax
from jax.experimental import pallas as pl
from jax.experimental.pallas import tpu as pltpu
from jax.experimental.pallas import tpu_sc as plsc

B, L, V, D = 4096, 200, 100000, 128
LP = 208
LANES = 16
NC, NS = 2, 16
NW = NC * NS
BPW = B // NW
LH = LP // 2
UNROLL = 16
NCH = D // LANES


def _sc_sums(emb, idxs2, maskf):
    mesh = plsc.VectorSubcoreMesh(core_axis_name="c", subcore_axis_name="s")

    @functools.partial(
        pl.kernel,
        out_type=jax.ShapeDtypeStruct((B, D), jnp.float32),
        mesh=mesh,
        scratch_types=[
            pltpu.VMEM((2 * BPW, LH), jnp.int32),
            pltpu.VMEM((LP,), jnp.float32),
            pltpu.VMEM((LP,), jnp.float32),
            pltpu.VMEM((LP, D), jnp.float32),
            pltpu.VMEM((LP, D), jnp.float32),
            pltpu.VMEM((BPW, D), jnp.float32),
            pltpu.SemaphoreType.DMA,
            pltpu.SemaphoreType.DMA,
        ],
    )
    def k(emb_hbm, idx_hbm, mask_hbm, out_hbm,
          idx_v, mrow0, mrow1, rows0, rows1, out_v, sem0, sem1):
        wid = lax.axis_index("s") * NC + lax.axis_index("c")
        base = wid * BPW
        pltpu.sync_copy(idx_hbm.at[pl.ds(2 * base, 2 * BPW)], idx_v)

        def issue(b, rows, mrow, sem):
            pltpu.async_copy(emb_hbm.at[idx_v.at[2 * b]],
                             rows.at[pl.ds(0, LH)], sem)
            pltpu.async_copy(emb_hbm.at[idx_v.at[2 * b + 1]],
                             rows.at[pl.ds(LH, LH)], sem)
            pltpu.async_copy(mask_hbm.at[base + b], mrow, sem)

        def drain(b, rows, mrow, sem):
            pltpu.make_async_copy(emb_hbm.at[idx_v.at[2 * b]],
                                  rows.at[pl.ds(0, LH)], sem).wait()
            pltpu.make_async_copy(emb_hbm.at[idx_v.at[2 * b + 1]],
                                  rows.at[pl.ds(LH, LH)], sem).wait()
            pltpu.make_async_copy(mask_hbm.at[base + b], mrow, sem).wait()

        def accum(b, rows, mrow):
            def rowstep(j, acc):
                r0 = j * UNROLL
                mv = mrow[pl.ds(r0, LANES)]
                for u in range(UNROLL):
                    mvec = jnp.full((LANES,), mv[u], jnp.float32)
                    acc = tuple(
                        acc[c] + rows[r0 + u, pl.ds(c * LANES, LANES)] * mvec
                        for c in range(NCH))
                return acc

            acc = lax.fori_loop(
                0, LP // UNROLL, rowstep,
                tuple(jnp.zeros((LANES,), jnp.float32) for _ in range(NCH)))
            for c in range(NCH):
                out_v[b, pl.ds(c * LANES, LANES)] = acc[c]

        issue(0, rows0, mrow0, sem0)

        def pair(g, carry):
            b0 = 2 * g
            b1 = 2 * g + 1
            issue(b1, rows1, mrow1, sem1)
            drain(b0, rows0, mrow0, sem0)
            accum(b0, rows0, mrow0)
            issue(lax.rem(b0 + 2, BPW), rows0, mrow0, sem0)
            drain(b1, rows1, mrow1, sem1)
            accum(b1, rows1, mrow1)
            return carry

        lax.fori_loop(0, BPW // 2, pair, 0)
        drain(0, rows0, mrow0, sem0)
        pltpu.sync_copy(out_v, out_hbm.at[pl.ds(base, BPW)])

    return k(emb, idxs2, maskf)


def _tc_finish(sums, mask, proj_t):
    BB = 512

    def body(s_ref, m_ref, p_ref, o_ref):
        cnt = jnp.sum(m_ref[...].astype(jnp.float32), axis=1, keepdims=True)
        y = jnp.dot(s_ref[...], p_ref[...], preferred_element_type=jnp.float32)
        o_ref[...] = y / jnp.clip(cnt, 1e-9, None)

    return pl.pallas_call(
        body,
        grid=(B // BB,),
        in_specs=[
            pl.BlockSpec((BB, D), lambda i: (i, 0)),
            pl.BlockSpec((BB, L), lambda i: (i, 0)),
            pl.BlockSpec((D, D), lambda i: (0, 0)),
        ],
        out_specs=pl.BlockSpec((BB, D), lambda i: (i, 0)),
        out_shape=jax.ShapeDtypeStruct((B, D), jnp.float32),
    )(sums, mask, proj_t)


def kernel(input_ids, attention_mask, emb, proj):
    ids = input_ids.astype(jnp.int32)
    msk = attention_mask.astype(jnp.int32)
    mskp = jnp.pad(msk, ((0, 0), (0, LP - L)))
    spread = (jnp.arange(B * LP, dtype=jnp.int32) % V).reshape(B, LP)
    idxs = jnp.where(mskp == 1, jnp.pad(ids, ((0, 0), (0, LP - L))), spread)
    sums = _sc_sums(emb, idxs.reshape(2 * B, LH), mskp.astype(jnp.float32))
    return _tc_finish(sums, msk, proj.T)


# speedup vs baseline: 66.3537x; 1.0018x over previous
import functools

import jax
import jax.numpy as jnp
from jax import lax
from jax.experimental import pallas as pl
from jax.experimental.pallas import tpu as pltpu
from jax.experimental.pallas import tpu_sc as plsc

B, L, V, D = 4096, 200, 100000, 128
LP = 208
LANES = 16
NC, NS = 2, 16
NW = NC * NS
BPW = B // NW
LH = LP // 2
UNROLL = 16
NCH = D // LANES


def _sc_sums(emb, idxs2, maskf):
    mesh = plsc.VectorSubcoreMesh(core_axis_name="c", subcore_axis_name="s")

    @functools.partial(
        pl.kernel,
        out_type=jax.ShapeDtypeStruct((B, D), jnp.float32),
        mesh=mesh,
        scratch_types=[
            pltpu.VMEM((2 * BPW, LH), jnp.int32),
            pltpu.VMEM((LP,), jnp.float32),
            pltpu.VMEM((LP,), jnp.float32),
            pltpu.VMEM((LP, D), jnp.float32),
            pltpu.VMEM((LP, D), jnp.float32),
            pltpu.VMEM((BPW, D), jnp.float32),
            pltpu.SemaphoreType.DMA,
            pltpu.SemaphoreType.DMA,
        ],
    )
    def k(emb_hbm, idx_hbm, mask_hbm, out_hbm,
          idx_v, mrow0, mrow1, rows0, rows1, out_v, sem0, sem1):
        wid = lax.axis_index("s") * NC + lax.axis_index("c")
        base = wid * BPW
        pltpu.sync_copy(idx_hbm.at[pl.ds(2 * base, 2 * BPW)], idx_v)

        SPLITS = ((0, 64), (64, LH - 64))

        def issue(b, rows, mrow, sem):
            for h in range(2):
                for (o, n) in SPLITS:
                    pltpu.async_copy(emb_hbm.at[idx_v.at[2 * b + h, pl.ds(o, n)]],
                                     rows.at[pl.ds(h * LH + o, n)], sem)
            pltpu.async_copy(mask_hbm.at[base + b], mrow, sem)

        def drain(b, rows, mrow, sem):
            for h in range(2):
                for (o, n) in SPLITS:
                    pltpu.make_async_copy(
                        emb_hbm.at[idx_v.at[2 * b + h, pl.ds(o, n)]],
                        rows.at[pl.ds(h * LH + o, n)], sem).wait()
            pltpu.make_async_copy(mask_hbm.at[base + b], mrow, sem).wait()

        def accum(b, rows, mrow):
            def rowstep(j, acc):
                r0 = j * UNROLL
                mv = mrow[pl.ds(r0, LANES)]
                for u in range(UNROLL):
                    mvec = jnp.full((LANES,), mv[u], jnp.float32)
                    acc = tuple(
                        acc[c] + rows[r0 + u, pl.ds(c * LANES, LANES)] * mvec
                        for c in range(NCH))
                return acc

            acc = lax.fori_loop(
                0, LP // UNROLL, rowstep,
                tuple(jnp.zeros((LANES,), jnp.float32) for _ in range(NCH)))
            for c in range(NCH):
                out_v[b, pl.ds(c * LANES, LANES)] = acc[c]

        issue(0, rows0, mrow0, sem0)

        def pair(g, carry):
            b0 = 2 * g
            b1 = 2 * g + 1
            issue(b1, rows1, mrow1, sem1)
            drain(b0, rows0, mrow0, sem0)
            accum(b0, rows0, mrow0)
            issue(lax.rem(b0 + 2, BPW), rows0, mrow0, sem0)
            drain(b1, rows1, mrow1, sem1)
            accum(b1, rows1, mrow1)
            return carry

        lax.fori_loop(0, BPW // 2, pair, 0)
        drain(0, rows0, mrow0, sem0)
        pltpu.sync_copy(out_v, out_hbm.at[pl.ds(base, BPW)])

    return k(emb, idxs2, maskf)


def _tc_finish(sums, mask, proj_t):
    BB = 512

    def body(s_ref, m_ref, p_ref, o_ref):
        cnt = jnp.sum(m_ref[...].astype(jnp.float32), axis=1, keepdims=True)
        y = jnp.dot(s_ref[...], p_ref[...], preferred_element_type=jnp.float32)
        o_ref[...] = y / jnp.clip(cnt, 1e-9, None)

    return pl.pallas_call(
        body,
        grid=(B // BB,),
        in_specs=[
            pl.BlockSpec((BB, D), lambda i: (i, 0)),
            pl.BlockSpec((BB, L), lambda i: (i, 0)),
            pl.BlockSpec((D, D), lambda i: (0, 0)),
        ],
        out_specs=pl.BlockSpec((BB, D), lambda i: (i, 0)),
        out_shape=jax.ShapeDtypeStruct((B, D), jnp.float32),
    )(sums, mask, proj_t)


def kernel(input_ids, attention_mask, emb, proj):
    ids = input_ids.astype(jnp.int32)
    msk = attention_mask.astype(jnp.int32)
    mskp = jnp.pad(msk, ((0, 0), (0, LP - L)))
    spread = (jnp.arange(B * LP, dtype=jnp.int32) % V).reshape(B, LP)
    idxs = jnp.where(mskp == 1, jnp.pad(ids, ((0, 0), (0, LP - L))), spread)
    sums = _sc_sums(emb, idxs.reshape(2 * B, LH), mskp.astype(jnp.float32))
    return _tc_finish(sums, msk, proj.T)
